# baseline (speedup 1.0000x reference)
import functools
import jax
import jax.numpy as jnp
from jax import lax
from jax.experimental import pallas as pl
from jax.experimental.pallas import tpu as pltpu
from jax.experimental.pallas import tpu_sc as plsc

_N = 1000
_ROWS = 16384
_L = 16
_NW = 32
_RPW = _ROWS // _NW
_NB = _RPW // _L
_NC = _N // _L
_CAND = 5 * _L
_STR = 85
_NEG = -1e7


def _insert5(t, v):
    t1, t2, t3, t4, t5 = t
    m = jnp.maximum(t1, v); v = jnp.minimum(t1, v); t1 = m
    m = jnp.maximum(t2, v); v = jnp.minimum(t2, v); t2 = m
    m = jnp.maximum(t3, v); v = jnp.minimum(t3, v); t3 = m
    m = jnp.maximum(t4, v); v = jnp.minimum(t4, v); t4 = m
    t5 = jnp.maximum(t5, v)
    return (t1, t2, t3, t4, t5)


def _make_sc_kernel():
    mesh = plsc.VectorSubcoreMesh(core_axis_name="c", subcore_axis_name="s")

    @functools.partial(
        pl.kernel,
        mesh=mesh,
        out_type=jax.ShapeDtypeStruct((_ROWS,), jnp.float32),
        scratch_types=[
            pltpu.VMEM((_L, _N), jnp.float32),
            pltpu.VMEM((_L, _N), jnp.float32),
            pltpu.VMEM((_L * _STR,), jnp.float32),
            pltpu.VMEM((_RPW,), jnp.int32),
            pltpu.VMEM((_RPW,), jnp.float32),
            pltpu.SemaphoreType.DMA,
            pltpu.SemaphoreType.DMA,
        ],
        compiler_params=pltpu.CompilerParams(use_tc_tiling_on_sc=True,
                                             needs_layout_passes=False,
                                             disable_bounds_checks=True,
                                             disable_semaphore_checks=True,
                                             skip_device_barrier=True),
    )
    def _sc_kernel(logits_hbm, labels_hbm, out_hbm, buf0, buf1, cand,
                   labs_v, out_v, sem0, sem1):
        wid = lax.axis_index("s") * 2 + lax.axis_index("c")
        base = wid * _RPW
        pltpu.sync_copy(labels_hbm.at[pl.ds(base, _RPW)], labs_v)

        lane = lax.broadcasted_iota(jnp.int32, (_L,), 0)
        ninf = jnp.full((_L,), -jnp.inf, jnp.float32)
        tail_idx = jnp.minimum(jnp.full((_L,), _NC * _L, jnp.int32) + lane,
                               _N - 1)
        tail_msk = lane < (_N - _NC * _L)
        cand_base = lane * _STR

        def start_copy(b, buf, sem):
            src = logits_hbm.at[pl.ds(base + b * _L, _L)]
            pltpu.async_copy(src, buf, sem)

        def wait_copy(b, buf, sem):
            src = logits_hbm.at[pl.ds(base + b * _L, _L)]
            pltpu.make_async_copy(src, buf, sem).wait()

        def compute(b, buf):
            labs = labs_v[pl.ds(b * _L, _L)]
            true_v = plsc.load_gather(buf, [lane, labs])
            plsc.store_scatter(buf, [lane, labs],
                               jnp.full((_L,), _NEG, jnp.float32))

            def row_body(rr, carry):
                r0 = 4 * rr
                r1, r2, r3 = r0 + 1, r0 + 2, r0 + 3

                @plsc.parallel_loop(0, _NC, step=1, unroll=1,
                                    carry=((ninf,) * 5,) * 4)
                def chunks(j, t):
                    ta, tb, tc, td = t
                    ta = _insert5(ta, buf[r0, pl.ds(j * _L, _L)])
                    tb = _insert5(tb, buf[r1, pl.ds(j * _L, _L)])
                    tc = _insert5(tc, buf[r2, pl.ds(j * _L, _L)])
                    td = _insert5(td, buf[r3, pl.ds(j * _L, _L)])
                    return (ta, tb, tc, td)

                ta, tb, tc, td = chunks
                for r, t in ((r0, ta), (r1, tb), (r2, tc), (r3, td)):
                    vt = plsc.load_gather(
                        buf, [jnp.full((_L,), r, jnp.int32), tail_idx])
                    vt = jnp.where(tail_msk, vt, -jnp.inf)
                    t1, t2, t3, t4, t5 = _insert5(t, vt)
                    rb = r * _STR
                    plsc.store_scatter(cand, [rb + lane], t1)
                    plsc.store_scatter(cand, [rb + _L + lane], t2)
                    plsc.store_scatter(cand, [rb + 2 * _L + lane], t3)
                    plsc.store_scatter(cand, [rb + 3 * _L + lane], t4)
                    plsc.store_scatter(cand, [rb + 4 * _L + lane], t5)
                return carry

            lax.fori_loop(0, _L // 4, row_body, 0)

            @plsc.parallel_loop(0, _CAND, step=1, unroll=4,
                                carry=((ninf,) * 5, cand_base))
            def ph2(c, t):
                tt, idx = t
                v = plsc.load_gather(cand, [idx])
                return (_insert5(tt, v), idx + 1)

            (t1, t2, t3, t4, t5), _ = ph2
            base_m = true_v + 1.0
            loss = jnp.maximum(base_m - t1, 0.0)
            loss = loss + jnp.maximum(base_m - t2, 0.0)
            loss = loss + jnp.maximum(base_m - t3, 0.0)
            loss = loss + jnp.maximum(base_m - t4, 0.0)
            loss = loss + jnp.maximum(base_m - t5, 0.0)
            out_v[pl.ds(b * _L, _L)] = loss

        start_copy(0, buf0, sem0)

        def outer(i, carry):
            b = 2 * i
            start_copy(b + 1, buf1, sem1)
            wait_copy(b, buf0, sem0)
            compute(b, buf0)

            @pl.when(b + 2 < _NB)
            def _():
                start_copy(b + 2, buf0, sem0)

            wait_copy(b + 1, buf1, sem1)
            compute(b + 1, buf1)
            return carry

        lax.fori_loop(0, _NB // 2, outer, 0)
        pltpu.sync_copy(out_v, out_hbm.at[pl.ds(base, _RPW)])

    return _sc_kernel


_SC_KERNEL = _make_sc_kernel()


def kernel(logits, labels):
    return _SC_KERNEL(logits, labels.astype(jnp.int32))


# speedup vs baseline: 1.0066x; 1.0066x over previous
import functools
import jax
import jax.numpy as jnp
from jax import lax
from jax.experimental import pallas as pl
from jax.experimental.pallas import tpu as pltpu
from jax.experimental.pallas import tpu_sc as plsc

_N = 1000
_ROWS = 16384
_L = 16
_NW = 32
_NC = _N // _L
_CAND = 5 * _L
_STR = 85
_NEG = -1e7

_SC_ROWS = 7168
_TC_ROWS = _ROWS - _SC_ROWS
_BR = 256


def _insert5(t, v):
    t1, t2, t3, t4, t5 = t
    m = jnp.maximum(t1, v); v = jnp.minimum(t1, v); t1 = m
    m = jnp.maximum(t2, v); v = jnp.minimum(t2, v); t2 = m
    m = jnp.maximum(t3, v); v = jnp.minimum(t3, v); t3 = m
    m = jnp.maximum(t4, v); v = jnp.minimum(t4, v); t4 = m
    t5 = jnp.maximum(t5, v)
    return (t1, t2, t3, t4, t5)


def _make_sc_kernel(nrows):
    rpw = nrows // _NW
    nb = rpw // _L
    assert nb % 2 == 0 and nb * _L * _NW == nrows
    mesh = plsc.VectorSubcoreMesh(core_axis_name="c", subcore_axis_name="s")

    @functools.partial(
        pl.kernel,
        mesh=mesh,
        out_type=jax.ShapeDtypeStruct((nrows,), jnp.float32),
        scratch_types=[
            pltpu.VMEM((_L, _N), jnp.float32),
            pltpu.VMEM((_L, _N), jnp.float32),
            pltpu.VMEM((_L * _STR,), jnp.float32),
            pltpu.VMEM((rpw,), jnp.int32),
            pltpu.VMEM((rpw,), jnp.float32),
            pltpu.SemaphoreType.DMA,
            pltpu.SemaphoreType.DMA,
        ],
        compiler_params=pltpu.CompilerParams(use_tc_tiling_on_sc=True,
                                             needs_layout_passes=False,
                                             disable_bounds_checks=True,
                                             disable_semaphore_checks=True,
                                             skip_device_barrier=True),
    )
    def _sc_kernel(logits_hbm, labels_hbm, out_hbm, buf0, buf1, cand,
                   labs_v, out_v, sem0, sem1):
        wid = lax.axis_index("s") * 2 + lax.axis_index("c")
        base = wid * rpw
        pltpu.sync_copy(labels_hbm.at[pl.ds(base, rpw)], labs_v)

        lane = lax.broadcasted_iota(jnp.int32, (_L,), 0)
        ninf = jnp.full((_L,), -jnp.inf, jnp.float32)
        tail_idx = jnp.minimum(jnp.full((_L,), _NC * _L, jnp.int32) + lane,
                               _N - 1)
        tail_msk = lane < (_N - _NC * _L)
        cand_base = lane * _STR

        def start_copy(b, buf, sem):
            src = logits_hbm.at[pl.ds(base + b * _L, _L)]
            pltpu.async_copy(src, buf, sem)

        def wait_copy(b, buf, sem):
            src = logits_hbm.at[pl.ds(base + b * _L, _L)]
            pltpu.make_async_copy(src, buf, sem).wait()

        def compute(b, buf):
            labs = labs_v[pl.ds(b * _L, _L)]
            true_v = plsc.load_gather(buf, [lane, labs])
            plsc.store_scatter(buf, [lane, labs],
                               jnp.full((_L,), _NEG, jnp.float32))

            def row_body(rr, carry):
                r0 = 2 * rr
                r1 = r0 + 1

                @plsc.parallel_loop(0, _NC, step=1, unroll=2,
                                    carry=((ninf,) * 5, (ninf,) * 5))
                def chunks(j, t):
                    ta, tb = t
                    ta = _insert5(ta, buf[r0, pl.ds(j * _L, _L)])
                    tb = _insert5(tb, buf[r1, pl.ds(j * _L, _L)])
                    return (ta, tb)

                ta, tb = chunks
                for r, t in ((r0, ta), (r1, tb)):
                    vt = plsc.load_gather(
                        buf, [jnp.full((_L,), r, jnp.int32), tail_idx])
                    vt = jnp.where(tail_msk, vt, -jnp.inf)
                    t1, t2, t3, t4, t5 = _insert5(t, vt)
                    rb = r * _STR
                    plsc.store_scatter(cand, [rb + lane], t1)
                    plsc.store_scatter(cand, [rb + _L + lane], t2)
                    plsc.store_scatter(cand, [rb + 2 * _L + lane], t3)
                    plsc.store_scatter(cand, [rb + 3 * _L + lane], t4)
                    plsc.store_scatter(cand, [rb + 4 * _L + lane], t5)
                return carry

            lax.fori_loop(0, _L // 2, row_body, 0)

            @plsc.parallel_loop(0, _CAND, step=1, unroll=4,
                                carry=((ninf,) * 5, cand_base))
            def ph2(c, t):
                tt, idx = t
                v = plsc.load_gather(cand, [idx])
                return (_insert5(tt, v), idx + 1)

            (t1, t2, t3, t4, t5), _ = ph2
            base_m = true_v + 1.0
            loss = jnp.maximum(base_m - t1, 0.0)
            loss = loss + jnp.maximum(base_m - t2, 0.0)
            loss = loss + jnp.maximum(base_m - t3, 0.0)
            loss = loss + jnp.maximum(base_m - t4, 0.0)
            loss = loss + jnp.maximum(base_m - t5, 0.0)
            out_v[pl.ds(b * _L, _L)] = loss

        start_copy(0, buf0, sem0)

        def outer(i, carry):
            b = 2 * i
            start_copy(b + 1, buf1, sem1)
            wait_copy(b, buf0, sem0)
            compute(b, buf0)

            @pl.when(b + 2 < nb)
            def _():
                start_copy(b + 2, buf0, sem0)

            wait_copy(b + 1, buf1, sem1)
            compute(b + 1, buf1)
            return carry

        lax.fori_loop(0, nb // 2, outer, 0)
        pltpu.sync_copy(out_v, out_hbm.at[pl.ds(base, rpw)])

    return _sc_kernel


def _tc_body(lab_ref, x_ref, out_ref):
    x = x_ref[...]
    lab = lab_ref[...]
    iota = lax.broadcasted_iota(jnp.int32, (_BR, _N), 1)
    onehot = iota == lab
    true1 = jnp.sum(jnp.where(onehot, x, 0.0), axis=1, keepdims=True)
    m = jnp.where(onehot, _NEG, x)
    loss = jnp.zeros((_BR, 1), jnp.float32)
    for _ in range(5):
        w = jnp.max(m, axis=1, keepdims=True)
        loss = loss + jnp.maximum(true1 - w + 1.0, 0.0)
        idx = jnp.min(jnp.where(m == w, iota, _N), axis=1, keepdims=True)
        m = jnp.where(iota == idx, -jnp.inf, m)
    out_ref[...] = loss


_SC_KERNEL = _make_sc_kernel(_SC_ROWS)

_ROW0 = _SC_ROWS // _BR


def _tc_part(logits, lab2):
    return pl.pallas_call(
        _tc_body,
        grid=(_TC_ROWS // _BR,),
        in_specs=[
            pl.BlockSpec((_BR, 1), lambda i: (i + _ROW0, 0)),
            pl.BlockSpec((_BR, _N), lambda i: (i + _ROW0, 0)),
        ],
        out_specs=pl.BlockSpec((_BR, 1), lambda i: (i, 0)),
        out_shape=jax.ShapeDtypeStruct((_TC_ROWS, 1), jnp.float32),
    )(lab2, logits)


def kernel(logits, labels):
    lab32 = labels.astype(jnp.int32)
    sc_out = _SC_KERNEL(logits, lab32)
    tc_out = _tc_part(logits, lab32.reshape(_ROWS, 1))
    return jnp.concatenate([sc_out, tc_out.reshape(_TC_ROWS)])


# speedup vs baseline: 2.1165x; 2.1026x over previous
import functools
import jax
import jax.numpy as jnp
from jax import lax
from jax.experimental import pallas as pl
from jax.experimental.pallas import tpu as pltpu
from jax.experimental.pallas import tpu_sc as plsc

_N = 1000
_ROWS = 16384
_L = 16
_NW = 32
_BRK = 128
_NEG = -1e7

_SC_ROWS = 8192
_TC_ROWS = _ROWS - _SC_ROWS
_BR = 256


def _insert5(t, v):
    t1, t2, t3, t4, t5 = t
    m = jnp.maximum(t1, v); v = jnp.minimum(t1, v); t1 = m
    m = jnp.maximum(t2, v); v = jnp.minimum(t2, v); t2 = m
    m = jnp.maximum(t3, v); v = jnp.minimum(t3, v); t3 = m
    m = jnp.maximum(t4, v); v = jnp.minimum(t4, v); t4 = m
    t5 = jnp.maximum(t5, v)
    return (t1, t2, t3, t4, t5)


def _make_sc_kernel(nrows):
    rpw = nrows // _NW
    nbk = rpw // _BRK
    assert nbk * _BRK * _NW == nrows
    mesh = plsc.VectorSubcoreMesh(core_axis_name="c", subcore_axis_name="s")

    @functools.partial(
        pl.kernel,
        mesh=mesh,
        out_type=jax.ShapeDtypeStruct((nrows,), jnp.float32),
        scratch_types=[
            pltpu.VMEM((_N, _BRK), jnp.float32),
            pltpu.VMEM((rpw,), jnp.int32),
            pltpu.VMEM((rpw,), jnp.float32),
        ],
        compiler_params=pltpu.CompilerParams(use_tc_tiling_on_sc=True,
                                             needs_layout_passes=False,
                                             disable_bounds_checks=True,
                                             disable_semaphore_checks=True,
                                             skip_device_barrier=True),
    )
    def _sc_kernel(logits_t_hbm, labels_hbm, out_hbm, buf, labs_v, out_v):
        wid = lax.axis_index("s") * 2 + lax.axis_index("c")
        base = wid * rpw
        pltpu.sync_copy(labels_hbm.at[pl.ds(base, rpw)], labs_v)

        lane = lax.broadcasted_iota(jnp.int32, (_L,), 0)
        ninf = jnp.full((_L,), -jnp.inf, jnp.float32)

        def block_body(b, carry):
            pltpu.sync_copy(
                logits_t_hbm.at[:, pl.ds(base + b * _BRK, _BRK)], buf)

            for g in range(_BRK // _L):
                off = b * _BRK + g * _L
                labs = labs_v[pl.ds(off, _L)]
                col = lane + g * _L
                true_v = plsc.load_gather(buf, [labs, col])
                plsc.store_scatter(buf, [labs, col],
                                   jnp.full((_L,), _NEG, jnp.float32))

                @plsc.parallel_loop(0, _N, step=1, unroll=2,
                                    carry=(ninf,) * 5)
                def cls_loop(c, t):
                    return _insert5(t, buf[c, pl.ds(g * _L, _L)])

                t1, t2, t3, t4, t5 = cls_loop
                base_m = true_v + 1.0
                loss = jnp.maximum(base_m - t1, 0.0)
                loss = loss + jnp.maximum(base_m - t2, 0.0)
                loss = loss + jnp.maximum(base_m - t3, 0.0)
                loss = loss + jnp.maximum(base_m - t4, 0.0)
                loss = loss + jnp.maximum(base_m - t5, 0.0)
                out_v[pl.ds(off, _L)] = loss
            return carry

        lax.fori_loop(0, nbk, block_body, 0)
        pltpu.sync_copy(out_v, out_hbm.at[pl.ds(base, rpw)])

    return _sc_kernel


def _tc_body(lab_ref, x_ref, out_ref):
    x = x_ref[...]
    lab = lab_ref[...][0]
    iota = lax.broadcasted_iota(jnp.int32, (_N, _BR), 0)
    onehot = iota == lab
    true1 = jnp.sum(jnp.where(onehot, x, 0.0), axis=0, keepdims=True)
    m = jnp.where(onehot, _NEG, x)
    loss = jnp.zeros((1, _BR), jnp.float32)
    for _ in range(5):
        w = jnp.max(m, axis=0, keepdims=True)
        loss = loss + jnp.maximum(true1 - w + 1.0, 0.0)
        idx = jnp.min(jnp.where(m == w, iota, _N), axis=0, keepdims=True)
        m = jnp.where(iota == idx, -jnp.inf, m)
    out_ref[...] = loss[jnp.newaxis]


_SC_KERNEL = _make_sc_kernel(_SC_ROWS)

_COL0 = _SC_ROWS // _BR


def _tc_part(logits_t, lab3):
    return pl.pallas_call(
        _tc_body,
        grid=(_TC_ROWS // _BR,),
        in_specs=[
            pl.BlockSpec((1, 1, _BR), lambda i: (i + _COL0, 0, 0)),
            pl.BlockSpec((_N, _BR), lambda i: (0, i + _COL0)),
        ],
        out_specs=pl.BlockSpec((1, 1, _BR), lambda i: (i, 0, 0)),
        out_shape=jax.ShapeDtypeStruct((_TC_ROWS // _BR, 1, _BR),
                                       jnp.float32),
    )(lab3, logits_t)


def kernel(logits, labels):
    lab32 = labels.astype(jnp.int32)
    logits_t = logits.T
    sc_out = _SC_KERNEL(logits_t, lab32)
    tc_out = _tc_part(logits_t, lab32.reshape(_ROWS // _BR, 1, _BR))
    return jnp.concatenate([sc_out, tc_out.reshape(_TC_ROWS)])
